# Initial kernel scaffold; baseline (speedup 1.0000x reference)
import jax
import jax.numpy as jnp
from jax.experimental import pallas as pl


def kernel(imgs, grads, reg_lab):
    raise NotImplementedError("write your pallas kernel here")


# speedup vs baseline: 31.6777x; 31.6777x over previous
import functools
import jax
import jax.numpy as jnp
from jax import lax
from jax.experimental import pallas as pl
from jax.experimental.pallas import tpu as pltpu
from jax.experimental.pallas import tpu_sc as plsc

_S = 1024
_CB = 32
_TB = 8
_B = 4
_C = 3
_G = 8
_HW = 512 * 512
_HALF = _HW // 2
_CHUNK = 4096
_NCHUNK = _HALF // _CHUNK
_NW = 32
_TPW = 7

_mesh = plsc.VectorSubcoreMesh(core_axis_name="c", subcore_axis_name="s")


def _sc_hist_body(imgs_hbm, grads_hbm, rl_hbm, rs_hbm, ch_hbm, th_hbm,
                  hist_v, rl_v, val_v):
    cid = lax.axis_index("c")
    sid = lax.axis_index("s")
    wid = cid * 16 + sid

    zeros16 = jnp.zeros((16,), jnp.float32)
    ones16 = jnp.ones((16,), jnp.float32)

    def zero_hist(nbins):
        def zb(i, _):
            base = i * 128
            for u in range(8):
                hist_v[pl.ds(base + u * 16, 16)] = zeros16
            return ()
        lax.fori_loop(0, nbins // 128, zb, ())

    def accum_chunk(scale, mult):
        def jb(j, _):
            base = j * 128
            for u in range(8):
                off = base + u * 16
                r = rl_v[pl.ds(off, 16)]
                if mult is None:
                    keys = r
                else:
                    v = val_v[pl.ds(off, 16)]
                    bins = (v * scale).astype(jnp.int32)
                    keys = r * mult + bins
                plsc.addupdate_scatter(hist_v, [keys], ones16)
            return ()
        lax.fori_loop(0, _CHUNK // 128, jb, ())

    def run(b, row, h, val_hbm, nbins, scale, mult):
        zero_hist(nbins)
        def cb(ci, _):
            start = h * _HALF + ci * _CHUNK
            pltpu.sync_copy(rl_hbm.at[b, pl.ds(start, _CHUNK)], rl_v)
            if val_hbm is not None:
                pltpu.sync_copy(val_hbm.at[row, pl.ds(start, _CHUNK)], val_v)
            accum_chunk(scale, mult)
            return ()
        lax.fori_loop(0, _NCHUNK, cb, ())

    def body(k, _):
        t = wid * _TPW + k

        @pl.when(t < 8)
        def _():
            b = t // 2
            h = t % 2
            run(b, 0, h, None, _S, 0.0, None)
            pltpu.sync_copy(hist_v.at[pl.ds(0, _S)], rs_hbm.at[h, b])

        @pl.when(jnp.logical_and(t >= 8, t < 32))
        def _():
            u = t - 8
            row = u // 2
            h = u % 2
            b = row // _C
            run(b, row, h, imgs_hbm, _S * _CB, float(_CB - 1), _CB)
            pltpu.sync_copy(hist_v, ch_hbm.at[h, row])

        @pl.when(t >= 32)
        def _():
            u = t - 32
            row = u // 2
            h = u % 2
            b = row // (_C * _G)
            run(b, row, h, grads_hbm, _S * _TB, float(_TB - 1), _TB)
            pltpu.sync_copy(hist_v.at[pl.ds(0, _S * _TB)], th_hbm.at[h, row])

        return ()

    lax.fori_loop(0, _TPW, body, ())


_sc_hist = functools.partial(
    pl.kernel,
    out_type=(
        jax.ShapeDtypeStruct((2, _B, _S), jnp.float32),
        jax.ShapeDtypeStruct((2, _B * _C, _S * _CB), jnp.float32),
        jax.ShapeDtypeStruct((2, _B * _C * _G, _S * _TB), jnp.float32),
    ),
    mesh=_mesh,
    scratch_types=[
        pltpu.VMEM((_S * _CB,), jnp.float32),
        pltpu.VMEM((_CHUNK,), jnp.int32),
        pltpu.VMEM((_CHUNK,), jnp.float32),
    ],
    compiler_params=pltpu.CompilerParams(needs_layout_passes=False),
)(_sc_hist_body)


def kernel(imgs, grads, reg_lab):
    B, C = imgs.shape[0], imgs.shape[1]
    G = grads.shape[2]
    imgs2 = imgs.reshape(B * C, _HW)
    grads2 = grads.reshape(B * C * G, _HW)
    rl2 = reg_lab.astype(jnp.int32).reshape(B, _HW)

    rs2, ch2, th2 = _sc_hist(imgs2, grads2, rl2)

    rs = rs2.sum(axis=0)
    ch = ch2.sum(axis=0).reshape(B, C, _S, _CB)
    ch = ch.transpose(0, 2, 1, 3).reshape(B, _S, C * _CB)
    chs = ch.sum(axis=-1, keepdims=True)
    ch = ch / jnp.where(chs > 0, chs, 1.0)
    th = th2.sum(axis=0).reshape(B, C, G, _S, _TB)
    th = th.transpose(0, 3, 1, 2, 4).reshape(B, _S, C * G * _TB)
    ths = th.sum(axis=-1, keepdims=True)
    th = th / jnp.where(ths > 0, ths, 1.0)
    return jnp.concatenate([rs, ch.reshape(B, -1), th.reshape(B, -1)], axis=-1)


# speedup vs baseline: 42.4062x; 1.3387x over previous
import functools
import jax
import jax.numpy as jnp
from jax import lax
from jax.experimental import pallas as pl
from jax.experimental.pallas import tpu as pltpu
from jax.experimental.pallas import tpu_sc as plsc

_S = 1024
_CB = 32
_TB = 8
_B = 4
_C = 3
_G = 8
_HW = 512 * 512
_HALF = _HW // 2
_CHUNK = 16384
_NCHUNK = _HALF // _CHUNK
_NW = 32
_TPW = 7

_mesh = plsc.VectorSubcoreMesh(core_axis_name="c", subcore_axis_name="s")


def _sc_hist_body(imgs_hbm, grads_hbm, rl_hbm, rs_hbm, ch_hbm, th_hbm,
                  hist_v, rl0_v, rl1_v, val0_v, val1_v, sem0, sem1):
    cid = lax.axis_index("c")
    sid = lax.axis_index("s")
    wid = cid * 16 + sid

    zeros16 = jnp.zeros((16,), jnp.float32)
    ones16 = jnp.ones((16,), jnp.float32)

    def zero_hist(nbins):
        def zb(i, _):
            base = i * 128
            for u in range(8):
                hist_v[pl.ds(base + u * 16, 16)] = zeros16
            return ()
        lax.fori_loop(0, nbins // 128, zb, ())

    bufs = [(rl0_v, val0_v, sem0), (rl1_v, val1_v, sem1)]

    def accum_chunk(rl_v, val_v, scale, mult):
        def jb(j, _):
            base = j * 128
            for u in range(8):
                off = base + u * 16
                r = rl_v[pl.ds(off, 16)]
                if mult is None:
                    keys = r
                else:
                    v = val_v[pl.ds(off, 16)]
                    bins = (v * scale).astype(jnp.int32)
                    keys = r * mult + bins
                plsc.addupdate_scatter(hist_v, [keys], ones16)
            return ()
        lax.fori_loop(0, _CHUNK // 128, jb, ())

    def run(b, row, h, val_hbm, nbins, scale, mult):
        zero_hist(nbins)

        def start_dma(ci):
            rl_v, val_v, sem = bufs[ci % 2]
            st = h * _HALF + ci * _CHUNK
            cps = [pltpu.async_copy(rl_hbm.at[b, pl.ds(st, _CHUNK)], rl_v, sem)]
            if val_hbm is not None:
                cps.append(
                    pltpu.async_copy(val_hbm.at[row, pl.ds(st, _CHUNK)], val_v, sem))
            return cps

        pend = start_dma(0)
        for ci in range(_NCHUNK):
            for cp in pend:
                cp.wait()
            if ci + 1 < _NCHUNK:
                pend = start_dma(ci + 1)
            rl_v, val_v, _ = bufs[ci % 2]
            accum_chunk(rl_v, val_v, scale, mult)

    def body(k, _):
        t = wid * _TPW + k

        @pl.when(t < 8)
        def _():
            b = t // 2
            h = t % 2
            run(b, 0, h, None, _S, 0.0, None)
            pltpu.sync_copy(hist_v.at[pl.ds(0, _S)], rs_hbm.at[h, b])

        @pl.when(jnp.logical_and(t >= 8, t < 32))
        def _():
            u = t - 8
            row = u // 2
            h = u % 2
            b = row // _C
            run(b, row, h, imgs_hbm, _S * _CB, float(_CB - 1), _CB)
            pltpu.sync_copy(hist_v, ch_hbm.at[h, row])

        @pl.when(t >= 32)
        def _():
            u = t - 32
            row = u // 2
            h = u % 2
            b = row // (_C * _G)
            run(b, row, h, grads_hbm, _S * _TB, float(_TB - 1), _TB)
            pltpu.sync_copy(hist_v.at[pl.ds(0, _S * _TB)], th_hbm.at[h, row])

        return ()

    lax.fori_loop(0, _TPW, body, ())


_sc_hist = functools.partial(
    pl.kernel,
    out_type=(
        jax.ShapeDtypeStruct((2, _B, _S), jnp.float32),
        jax.ShapeDtypeStruct((2, _B * _C, _S * _CB), jnp.float32),
        jax.ShapeDtypeStruct((2, _B * _C * _G, _S * _TB), jnp.float32),
    ),
    mesh=_mesh,
    scratch_types=[
        pltpu.VMEM((_S * _CB,), jnp.float32),
        pltpu.VMEM((_CHUNK,), jnp.int32),
        pltpu.VMEM((_CHUNK,), jnp.int32),
        pltpu.VMEM((_CHUNK,), jnp.float32),
        pltpu.VMEM((_CHUNK,), jnp.float32),
        pltpu.SemaphoreType.DMA,
        pltpu.SemaphoreType.DMA,
    ],
    compiler_params=pltpu.CompilerParams(needs_layout_passes=False),
)(_sc_hist_body)


def kernel(imgs, grads, reg_lab):
    B, C = imgs.shape[0], imgs.shape[1]
    G = grads.shape[2]
    imgs2 = imgs.reshape(B * C, _HW)
    grads2 = grads.reshape(B * C * G, _HW)
    rl2 = reg_lab.astype(jnp.int32).reshape(B, _HW)

    rs2, ch2, th2 = _sc_hist(imgs2, grads2, rl2)

    rs = rs2.sum(axis=0)
    ch = ch2.sum(axis=0).reshape(B, C, _S, _CB)
    ch = ch.transpose(0, 2, 1, 3).reshape(B, _S, C * _CB)
    chs = ch.sum(axis=-1, keepdims=True)
    ch = ch / jnp.where(chs > 0, chs, 1.0)
    th = th2.sum(axis=0).reshape(B, C, G, _S, _TB)
    th = th.transpose(0, 3, 1, 2, 4).reshape(B, _S, C * G * _TB)
    ths = th.sum(axis=-1, keepdims=True)
    th = th / jnp.where(ths > 0, ths, 1.0)
    return jnp.concatenate([rs, ch.reshape(B, -1), th.reshape(B, -1)], axis=-1)


# speedup vs baseline: 77.3804x; 1.8247x over previous
import functools
import jax
import jax.numpy as jnp
from jax import lax
from jax.experimental import pallas as pl
from jax.experimental.pallas import tpu as pltpu
from jax.experimental.pallas import tpu_sc as plsc

_S = 1024
_CB = 32
_TB = 8
_B = 4
_C = 3
_G = 8
_HW = 512 * 512
_HALF = _HW // 2
_CHUNK = 16384
_NCHUNK = _HALF // _CHUNK
_NW = 32
_TPW = 7

_mesh = plsc.VectorSubcoreMesh(core_axis_name="c", subcore_axis_name="s")


def _sc_hist_body(imgs_hbm, grads_hbm, rl_hbm, rs_hbm, ch_hbm, th_hbm,
                  hist_v, rl0_v, rl1_v, val0_v, val1_v, sem0, sem1):
    cid = lax.axis_index("c")
    sid = lax.axis_index("s")
    wid = cid * 16 + sid

    zeros16 = jnp.zeros((16,), jnp.float32)
    ones16 = jnp.ones((16,), jnp.float32)

    def zero_hist(nbins):
        def zb(i, _):
            base = i * 128
            for u in range(8):
                hist_v[pl.ds(base + u * 16, 16)] = zeros16
            return ()
        lax.fori_loop(0, nbins // 128, zb, ())

    bufs = [(rl0_v, val0_v, sem0), (rl1_v, val1_v, sem1)]

    def accum_chunk(rl_v, val_v, scale, mult):
        def jb(j, _):
            base = j * 128
            keyvecs = []
            for u in range(8):
                off = base + u * 16
                r = rl_v[pl.ds(off, 16)]
                if mult is None:
                    keyvecs.append(r)
                else:
                    v = val_v[pl.ds(off, 16)]
                    bins = (v * scale).astype(jnp.int32)
                    keyvecs.append(r * mult + bins)
            for keys in keyvecs:
                plsc.addupdate_scatter(hist_v, [keys], ones16)
            return ()
        lax.fori_loop(0, _CHUNK // 128, jb, ())

    def run(b, row, h, val_hbm, nbins, scale, mult):
        zero_hist(nbins)

        def start_dma(ci):
            rl_v, val_v, sem = bufs[ci % 2]
            st = h * _HALF + ci * _CHUNK
            cps = [pltpu.async_copy(rl_hbm.at[b, pl.ds(st, _CHUNK)], rl_v, sem)]
            if val_hbm is not None:
                cps.append(
                    pltpu.async_copy(val_hbm.at[row, pl.ds(st, _CHUNK)], val_v, sem))
            return cps

        pend = start_dma(0)
        for ci in range(_NCHUNK):
            for cp in pend:
                cp.wait()
            if ci + 1 < _NCHUNK:
                pend = start_dma(ci + 1)
            rl_v, val_v, _ = bufs[ci % 2]
            accum_chunk(rl_v, val_v, scale, mult)

    def body(k, _):
        t = wid * _TPW + k

        @pl.when(t < 8)
        def _():
            b = t // 2
            h = t % 2
            run(b, 0, h, None, _S, 0.0, None)
            pltpu.sync_copy(hist_v.at[pl.ds(0, _S)], rs_hbm.at[h, b])

        @pl.when(jnp.logical_and(t >= 8, t < 32))
        def _():
            u = t - 8
            row = u // 2
            h = u % 2
            b = row // _C
            run(b, row, h, imgs_hbm, _S * _CB, float(_CB - 1), _CB)
            pltpu.sync_copy(hist_v, ch_hbm.at[h, row])

        @pl.when(t >= 32)
        def _():
            u = t - 32
            row = u // 2
            h = u % 2
            b = row // (_C * _G)
            run(b, row, h, grads_hbm, _S * _TB, float(_TB - 1), _TB)
            pltpu.sync_copy(hist_v.at[pl.ds(0, _S * _TB)], th_hbm.at[h, row])

        return ()

    lax.fori_loop(0, _TPW, body, ())


_sc_hist = functools.partial(
    pl.kernel,
    out_type=(
        jax.ShapeDtypeStruct((2, _B, _S), jnp.float32),
        jax.ShapeDtypeStruct((2, _B * _C, _S * _CB), jnp.float32),
        jax.ShapeDtypeStruct((2, _B * _C * _G, _S * _TB), jnp.float32),
    ),
    mesh=_mesh,
    scratch_types=[
        pltpu.VMEM((_S * _CB,), jnp.float32),
        pltpu.VMEM((_CHUNK,), jnp.int32),
        pltpu.VMEM((_CHUNK,), jnp.int32),
        pltpu.VMEM((_CHUNK,), jnp.float32),
        pltpu.VMEM((_CHUNK,), jnp.float32),
        pltpu.SemaphoreType.DMA,
        pltpu.SemaphoreType.DMA,
    ],
    compiler_params=pltpu.CompilerParams(needs_layout_passes=False),
)(_sc_hist_body)


def kernel(imgs, grads, reg_lab):
    B, C = imgs.shape[0], imgs.shape[1]
    G = grads.shape[2]
    imgs2 = imgs.reshape(B * C, _HW)
    grads2 = grads.reshape(B * C * G, _HW)
    rl2 = reg_lab.astype(jnp.int32).reshape(B, _HW)

    rs2, ch2, th2 = _sc_hist(imgs2, grads2, rl2)

    rs = rs2.sum(axis=0)
    ch = ch2.sum(axis=0).reshape(B, C, _S, _CB)
    ch = ch.transpose(0, 2, 1, 3).reshape(B, _S, C * _CB)
    chs = ch.sum(axis=-1, keepdims=True)
    ch = ch / jnp.where(chs > 0, chs, 1.0)
    th = th2.sum(axis=0).reshape(B, C, G, _S, _TB)
    th = th.transpose(0, 3, 1, 2, 4).reshape(B, _S, C * G * _TB)
    ths = th.sum(axis=-1, keepdims=True)
    th = th / jnp.where(ths > 0, ths, 1.0)
    return jnp.concatenate([rs, ch.reshape(B, -1), th.reshape(B, -1)], axis=-1)


# speedup vs baseline: 80.6942x; 1.0428x over previous
import functools
import jax
import jax.numpy as jnp
from jax import lax
from jax.experimental import pallas as pl
from jax.experimental.pallas import tpu as pltpu
from jax.experimental.pallas import tpu_sc as plsc

_S = 1024
_CB = 32
_TB = 8
_B = 4
_C = 3
_G = 8
_HW = 512 * 512
_HALF = _HW // 2
_CHUNK = 8192
_NCHUNK = _HALF // _CHUNK
_TPW = 4
_UNROLL = 8

_mesh = plsc.VectorSubcoreMesh(core_axis_name="c", subcore_axis_name="s")


def _sc_hist_body(imgs_hbm, grads_hbm, rl_hbm, rs_hbm, ch_hbm, th_hbm,
                  hist_v, rl0_v, rl1_v, va0_v, va1_v, vb0_v, vb1_v,
                  sem0, sem1):
    cid = lax.axis_index("c")
    sid = lax.axis_index("s")
    wid = cid * 16 + sid

    zeros16 = jnp.zeros((16,), jnp.float32)
    ones16 = jnp.ones((16,), jnp.float32)

    bufs = [(rl0_v, va0_v, vb0_v, sem0), (rl1_v, va1_v, vb1_v, sem1)]

    def zero_hist(nbins):
        def zb(i, _):
            base = i * 128
            for u in range(8):
                hist_v[pl.ds(base + u * 16, 16)] = zeros16
            return ()
        lax.fori_loop(0, nbins // 128, zb, ())

    def run(b, row, h, kind):
        zero_hist({0: _S, 1: _S * _CB, 2: 2 * _S * _TB}[kind])

        def start_dma(ci):
            rl_v, va_v, vb_v, sem = bufs[ci % 2]
            st = h * _HALF + ci * _CHUNK
            cps = [pltpu.async_copy(rl_hbm.at[b, pl.ds(st, _CHUNK)], rl_v, sem)]
            if kind == 1:
                cps.append(
                    pltpu.async_copy(imgs_hbm.at[row, pl.ds(st, _CHUNK)], va_v, sem))
            elif kind == 2:
                cps.append(
                    pltpu.async_copy(grads_hbm.at[row, pl.ds(st, _CHUNK)], va_v, sem))
                cps.append(
                    pltpu.async_copy(grads_hbm.at[row + 1, pl.ds(st, _CHUNK)], vb_v, sem))
            return cps

        def accum_chunk(rl_v, va_v, vb_v):
            def jb(j, _):
                base = j * (16 * _UNROLL)
                keyvecs = []
                for u in range(_UNROLL):
                    off = base + u * 16
                    r = rl_v[pl.ds(off, 16)]
                    if kind == 0:
                        keyvecs.append(r)
                    elif kind == 1:
                        v = va_v[pl.ds(off, 16)]
                        keyvecs.append(r * _CB + (v * float(_CB - 1)).astype(jnp.int32))
                    else:
                        v0 = va_v[pl.ds(off, 16)]
                        v1 = vb_v[pl.ds(off, 16)]
                        rm = r * _TB
                        keyvecs.append(rm + (v0 * float(_TB - 1)).astype(jnp.int32))
                        keyvecs.append(
                            (rm + _S * _TB) + (v1 * float(_TB - 1)).astype(jnp.int32))
                for keys in keyvecs:
                    plsc.addupdate_scatter(hist_v, [keys], ones16)
                return ()
            lax.fori_loop(0, _CHUNK // (16 * _UNROLL), jb, ())

        pend = start_dma(0)
        for ci in range(_NCHUNK):
            for cp in pend:
                cp.wait()
            if ci + 1 < _NCHUNK:
                pend = start_dma(ci + 1)
            rl_v, va_v, vb_v, _ = bufs[ci % 2]
            accum_chunk(rl_v, va_v, vb_v)

    def body(k, _):
        t = wid * _TPW + k

        @pl.when(t < 8)
        def _():
            b = t // 2
            h = t % 2
            run(b, 0, h, 0)
            pltpu.sync_copy(hist_v.at[pl.ds(0, _S)], rs_hbm.at[h, b])

        @pl.when(jnp.logical_and(t >= 8, t < 32))
        def _():
            u = t - 8
            row = u // 2
            h = u % 2
            b = row // _C
            run(b, row, h, 1)
            pltpu.sync_copy(hist_v, ch_hbm.at[h, row])

        @pl.when(t >= 32)
        def _():
            u = t - 32
            pr = u // 2
            h = u % 2
            row0 = pr * 2
            b = pr // (_C * _G // 2)
            run(b, row0, h, 2)
            pltpu.sync_copy(hist_v.at[pl.ds(0, _S * _TB)], th_hbm.at[h, row0])
            pltpu.sync_copy(hist_v.at[pl.ds(_S * _TB, _S * _TB)],
                            th_hbm.at[h, row0 + 1])

        return ()

    lax.fori_loop(0, _TPW, body, ())


_sc_hist = functools.partial(
    pl.kernel,
    out_type=(
        jax.ShapeDtypeStruct((2, _B, _S), jnp.float32),
        jax.ShapeDtypeStruct((2, _B * _C, _S * _CB), jnp.float32),
        jax.ShapeDtypeStruct((2, _B * _C * _G, _S * _TB), jnp.float32),
    ),
    mesh=_mesh,
    scratch_types=[
        pltpu.VMEM((_S * _CB,), jnp.float32),
        pltpu.VMEM((_CHUNK,), jnp.int32),
        pltpu.VMEM((_CHUNK,), jnp.int32),
        pltpu.VMEM((_CHUNK,), jnp.float32),
        pltpu.VMEM((_CHUNK,), jnp.float32),
        pltpu.VMEM((_CHUNK,), jnp.float32),
        pltpu.VMEM((_CHUNK,), jnp.float32),
        pltpu.SemaphoreType.DMA,
        pltpu.SemaphoreType.DMA,
    ],
    compiler_params=pltpu.CompilerParams(needs_layout_passes=False),
)(_sc_hist_body)


def kernel(imgs, grads, reg_lab):
    B, C = imgs.shape[0], imgs.shape[1]
    G = grads.shape[2]
    imgs2 = imgs.reshape(B * C, _HW)
    grads2 = grads.reshape(B * C * G, _HW)
    rl2 = reg_lab.astype(jnp.int32).reshape(B, _HW)

    rs2, ch2, th2 = _sc_hist(imgs2, grads2, rl2)

    rs = rs2.sum(axis=0)
    ch = ch2.sum(axis=0).reshape(B, C, _S, _CB)
    ch = ch.transpose(0, 2, 1, 3).reshape(B, _S, C * _CB)
    chs = ch.sum(axis=-1, keepdims=True)
    ch = ch / jnp.where(chs > 0, chs, 1.0)
    th = th2.sum(axis=0).reshape(B, C, G, _S, _TB)
    th = th.transpose(0, 3, 1, 2, 4).reshape(B, _S, C * G * _TB)
    ths = th.sum(axis=-1, keepdims=True)
    th = th / jnp.where(ths > 0, ths, 1.0)
    return jnp.concatenate([rs, ch.reshape(B, -1), th.reshape(B, -1)], axis=-1)
